# baseline (speedup 1.0000x reference)
import jax
import jax.numpy as jnp
from jax.experimental import pallas as pl
from jax.experimental.pallas import tpu as pltpu

_BN = 1024
_BK = 2048


def _dist_kernel(e_ref, c_ref, o_ref):
    e = e_ref[...]
    c = c_ref[...]
    zc = jax.lax.dot_general(
        e, c, (((1,), (1,)), ((), ())),
        preferred_element_type=jnp.float32,
    )
    z2 = jnp.sum(e * e, axis=1, keepdims=True)
    c2 = jnp.sum(c * c, axis=1)[None, :]
    o_ref[...] = (z2 + c2) - 2.0 * zc


def kernel(encoding, codebook):
    n, d = encoding.shape
    k, _ = codebook.shape
    grid = (n // _BN, k // _BK)
    return pl.pallas_call(
        _dist_kernel,
        grid=grid,
        in_specs=[
            pl.BlockSpec((_BN, d), lambda i, j: (i, 0)),
            pl.BlockSpec((_BK, d), lambda i, j: (j, 0)),
        ],
        out_specs=pl.BlockSpec((_BN, _BK), lambda i, j: (i, j)),
        out_shape=jax.ShapeDtypeStruct((n, k), jnp.float32),
        compiler_params=pltpu.CompilerParams(
            dimension_semantics=("parallel", "arbitrary"),
        ),
    )(encoding, codebook)


# speedup vs baseline: 1.0022x; 1.0022x over previous
import jax
import jax.numpy as jnp
from jax.experimental import pallas as pl
from jax.experimental.pallas import tpu as pltpu

_BN = 1024
_BK = 2048


def _dist_kernel(e_ref, c_ref, o_ref):
    e = e_ref[...]
    c = c_ref[...]
    zc = jax.lax.dot_general(
        e.astype(jnp.bfloat16), c.astype(jnp.bfloat16),
        (((1,), (1,)), ((), ())),
        preferred_element_type=jnp.float32,
    )
    z2 = jnp.sum(e * e, axis=1, keepdims=True)
    c2 = jnp.sum(c * c, axis=1)[None, :]
    o_ref[...] = (z2 + c2) - 2.0 * zc


def kernel(encoding, codebook):
    n, d = encoding.shape
    k, _ = codebook.shape
    grid = (n // _BN, k // _BK)
    return pl.pallas_call(
        _dist_kernel,
        grid=grid,
        in_specs=[
            pl.BlockSpec((_BN, d), lambda i, j: (i, 0)),
            pl.BlockSpec((_BK, d), lambda i, j: (j, 0)),
        ],
        out_specs=pl.BlockSpec((_BN, _BK), lambda i, j: (i, j)),
        out_shape=jax.ShapeDtypeStruct((n, k), jnp.float32),
        compiler_params=pltpu.CompilerParams(
            dimension_semantics=("parallel", "arbitrary"),
        ),
    )(encoding, codebook)


# speedup vs baseline: 1.0391x; 1.0369x over previous
import jax
import jax.numpy as jnp
from jax.experimental import pallas as pl
from jax.experimental.pallas import tpu as pltpu

_BN = 1024
_BK = 2048
_D = 64
_DA = 68


def _aug_kernel(x_ref, o_ref, *, is_encoding):
    x = x_ref[...]
    n2 = jnp.sum(x * x, axis=1, keepdims=True)
    hi = n2.astype(jnp.bfloat16)
    lo = (n2 - hi.astype(jnp.float32)).astype(jnp.bfloat16)
    one = jnp.ones_like(hi)
    if is_encoding:
        cols = [(-2.0 * x).astype(jnp.bfloat16), hi, lo, one, one]
    else:
        cols = [x.astype(jnp.bfloat16), one, one, hi, lo]
    o_ref[...] = jnp.concatenate(cols, axis=1)


def _augment(x, is_encoding, bm):
    m = x.shape[0]
    bm = min(bm, m)
    return pl.pallas_call(
        lambda x_ref, o_ref: _aug_kernel(x_ref, o_ref, is_encoding=is_encoding),
        grid=(m // bm,),
        in_specs=[pl.BlockSpec((bm, _D), lambda i: (i, 0))],
        out_specs=pl.BlockSpec((bm, _DA), lambda i: (i, 0)),
        out_shape=jax.ShapeDtypeStruct((m, _DA), jnp.bfloat16),
    )(x)


def _dist_kernel(ea_ref, ca_ref, o_ref):
    o_ref[...] = jax.lax.dot_general(
        ea_ref[...], ca_ref[...], (((1,), (1,)), ((), ())),
        preferred_element_type=jnp.float32,
    )


def kernel(encoding, codebook):
    n, _ = encoding.shape
    k, _ = codebook.shape
    ea = _augment(encoding, True, 4096)
    ca = _augment(codebook, False, 4096)
    grid = (n // _BN, k // _BK)
    return pl.pallas_call(
        _dist_kernel,
        grid=grid,
        in_specs=[
            pl.BlockSpec((_BN, _DA), lambda i, j: (i, 0)),
            pl.BlockSpec((_BK, _DA), lambda i, j: (j, 0)),
        ],
        out_specs=pl.BlockSpec((_BN, _BK), lambda i, j: (i, j)),
        out_shape=jax.ShapeDtypeStruct((n, k), jnp.float32),
        compiler_params=pltpu.CompilerParams(
            dimension_semantics=("parallel", "arbitrary"),
        ),
    )(ea, ca)


# speedup vs baseline: 1.1229x; 1.0806x over previous
import jax
import jax.numpy as jnp
from jax.experimental import pallas as pl
from jax.experimental.pallas import tpu as pltpu

_BN = 256
_D = 64
_DA = 68


def _aug_kernel(x_ref, o_ref, *, is_encoding):
    x = x_ref[...]
    n2 = jnp.sum(x * x, axis=1, keepdims=True)
    hi = n2.astype(jnp.bfloat16)
    lo = (n2 - hi.astype(jnp.float32)).astype(jnp.bfloat16)
    one = jnp.ones_like(hi)
    if is_encoding:
        cols = [(-2.0 * x).astype(jnp.bfloat16), hi, lo, one, one]
    else:
        cols = [x.astype(jnp.bfloat16), one, one, hi, lo]
    o_ref[...] = jnp.concatenate(cols, axis=1)


def _augment(x, is_encoding, bm):
    m = x.shape[0]
    bm = min(bm, m)
    return pl.pallas_call(
        lambda x_ref, o_ref: _aug_kernel(x_ref, o_ref, is_encoding=is_encoding),
        grid=(m // bm,),
        in_specs=[pl.BlockSpec((bm, _D), lambda i: (i, 0))],
        out_specs=pl.BlockSpec((bm, _DA), lambda i: (i, 0)),
        out_shape=jax.ShapeDtypeStruct((m, _DA), jnp.bfloat16),
    )(x)


def _dist_kernel(ea_ref, ca_ref, o_ref):
    o_ref[...] = jax.lax.dot_general(
        ea_ref[...], ca_ref[...], (((1,), (1,)), ((), ())),
        preferred_element_type=jnp.float32,
    )


def kernel(encoding, codebook):
    n, _ = encoding.shape
    k, _ = codebook.shape
    ea = _augment(encoding, True, 4096)
    ca = _augment(codebook, False, 4096)
    grid = (n // _BN,)
    return pl.pallas_call(
        _dist_kernel,
        grid=grid,
        in_specs=[
            pl.BlockSpec((_BN, _DA), lambda i: (i, 0)),
            pl.BlockSpec((k, _DA), lambda i: (0, 0)),
        ],
        out_specs=pl.BlockSpec((_BN, k), lambda i: (i, 0)),
        out_shape=jax.ShapeDtypeStruct((n, k), jnp.float32),
        compiler_params=pltpu.CompilerParams(
            dimension_semantics=("arbitrary",),
        ),
    )(ea, ca)


# speedup vs baseline: 1.1291x; 1.0055x over previous
import jax
import jax.numpy as jnp
from jax.experimental import pallas as pl
from jax.experimental.pallas import tpu as pltpu

_BN = 512
_D = 64
_DA = 68


def _aug_kernel(x_ref, o_ref, *, is_encoding):
    x = x_ref[...]
    n2 = jnp.sum(x * x, axis=1, keepdims=True)
    hi = n2.astype(jnp.bfloat16)
    lo = (n2 - hi.astype(jnp.float32)).astype(jnp.bfloat16)
    one = jnp.ones_like(hi)
    if is_encoding:
        cols = [(-2.0 * x).astype(jnp.bfloat16), hi, lo, one, one]
    else:
        cols = [x.astype(jnp.bfloat16), one, one, hi, lo]
    o_ref[...] = jnp.concatenate(cols, axis=1)


def _augment(x, is_encoding, bm):
    m = x.shape[0]
    bm = min(bm, m)
    return pl.pallas_call(
        lambda x_ref, o_ref: _aug_kernel(x_ref, o_ref, is_encoding=is_encoding),
        grid=(m // bm,),
        in_specs=[pl.BlockSpec((bm, _D), lambda i: (i, 0))],
        out_specs=pl.BlockSpec((bm, _DA), lambda i: (i, 0)),
        out_shape=jax.ShapeDtypeStruct((m, _DA), jnp.bfloat16),
    )(x)


def _dist_kernel(ea_ref, ca_ref, o_ref):
    o_ref[...] = jax.lax.dot_general(
        ea_ref[...], ca_ref[...], (((1,), (1,)), ((), ())),
        preferred_element_type=jnp.float32,
    )


def kernel(encoding, codebook):
    n, _ = encoding.shape
    k, _ = codebook.shape
    ea = _augment(encoding, True, 4096)
    ca = _augment(codebook, False, 4096)
    grid = (n // _BN,)
    return pl.pallas_call(
        _dist_kernel,
        grid=grid,
        in_specs=[
            pl.BlockSpec((_BN, _DA), lambda i: (i, 0)),
            pl.BlockSpec((k, _DA), lambda i: (0, 0)),
        ],
        out_specs=pl.BlockSpec((_BN, k), lambda i: (i, 0)),
        out_shape=jax.ShapeDtypeStruct((n, k), jnp.float32),
        compiler_params=pltpu.CompilerParams(
            dimension_semantics=("arbitrary",),
        ),
    )(ea, ca)
